# baseline (speedup 1.0000x reference)
import functools

import jax
import jax.numpy as jnp
from jax import lax
from jax.experimental import pallas as pl
from jax.experimental.pallas import tpu as pltpu
from jax.experimental.pallas import tpu_sc as plsc

N = 10000
E = 320000
D = 128
K = 16

NC = 2
NS = 16
NW = NC * NS
B = 128
NBUF = 2
CHUNKS = 80
EPT = CHUNKS * B
EPAD = EPT * NW
DUMMY = N
NP = 10240
RPT = NP // NS
RB = 1024


_mesh = plsc.VectorSubcoreMesh(core_axis_name="c", subcore_axis_name="s")


@functools.partial(
    pl.kernel,
    mesh=_mesh,
    out_type=jax.ShapeDtypeStruct((NC, NP, D), jnp.float32),
    scratch_types=[
        pltpu.VMEM((B,), jnp.int32),
        pltpu.VMEM((B, D), jnp.float32),
        pltpu.VMEM((B, D), jnp.float32),
        pltpu.VMEM_SHARED((NP, D), jnp.float32),
        pltpu.SemaphoreType.DMA,
    ],
)
def _sc_hist(dst_hbm, out_hbm, idx_v, ones_v, zero_v, deg_sh, sem):
    c = lax.axis_index("c")
    s = lax.axis_index("s")

    def _fill(i, _):
        for j in range(D // 16):
            ones_v[i, pl.ds(j * 16, 16)] = jnp.ones((16,), jnp.float32)
            zero_v[i, pl.ds(j * 16, 16)] = jnp.zeros((16,), jnp.float32)
        return _

    lax.fori_loop(0, B, _fill, None)
    for k in range(RPT // B):
        pltpu.sync_copy(zero_v, deg_sh.at[pl.ds(s * RPT + k * B, B)])
    plsc.subcore_barrier()

    base = (c * NS + s) * EPT

    def _step(ch, _):
        pltpu.sync_copy(dst_hbm.at[pl.ds(base + ch * B, B)], idx_v)
        pltpu.sync_copy(ones_v, deg_sh.at[idx_v], add=True)
        return _

    lax.fori_loop(0, CHUNKS, _step, None)
    plsc.subcore_barrier()
    pltpu.sync_copy(deg_sh.at[pl.ds(s * RPT, RPT)],
                    out_hbm.at[c, pl.ds(s * RPT, RPT)])


@functools.partial(
    pl.kernel,
    mesh=_mesh,
    out_type=jax.ShapeDtypeStruct((NC, NP, D), jnp.float32),
    scratch_types=[
        pltpu.VMEM_SHARED((NP, D), jnp.float32),
    ]
    + [pltpu.VMEM((B,), jnp.int32) for _ in range(2 * NBUF)]
    + [pltpu.VMEM((B, D), jnp.float32) for _ in range(NBUF)]
    + [pltpu.SemaphoreType.DMA for _ in range(NBUF)],
)
def _sc_spmm(y_hbm, src_hbm, dst_hbm, out_hbm, acc_sh, *rest):
    src_v = rest[:NBUF]
    dst_v = rest[NBUF:2 * NBUF]
    rows = rest[2 * NBUF:3 * NBUF]
    gsem = rest[3 * NBUF:4 * NBUF]
    c = lax.axis_index("c")
    s = lax.axis_index("s")
    base = (c * NS + s) * EPT

    def _zero(i, _):
        for j in range(D // 16):
            rows[0][i, pl.ds(j * 16, 16)] = jnp.zeros((16,), jnp.float32)
        return _

    lax.fori_loop(0, B, _zero, None)
    for k in range(RPT // B):
        pltpu.sync_copy(rows[0], acc_sh.at[pl.ds(s * RPT + k * B, B)])
    plsc.subcore_barrier()

    def _sidx(ch):
        return src_hbm.at[pl.ds(base + ch * B, B)]

    def _didx(ch):
        return dst_hbm.at[pl.ds(base + ch * B, B)]

    def _pair(g, _):
        ch0 = g * 2
        pltpu.sync_copy(_sidx(ch0), src_v[0])
        pltpu.sync_copy(_didx(ch0), dst_v[0])
        d0 = pltpu.async_copy(y_hbm.at[src_v[0]], rows[0], gsem[0])
        pltpu.sync_copy(_sidx(ch0 + 1), src_v[1])
        pltpu.sync_copy(_didx(ch0 + 1), dst_v[1])
        d0.wait()
        d1 = pltpu.async_copy(y_hbm.at[src_v[1]], rows[1], gsem[1])
        pltpu.sync_copy(rows[0], acc_sh.at[dst_v[0]], add=True)
        d1.wait()
        pltpu.sync_copy(rows[1], acc_sh.at[dst_v[1]], add=True)
        return _

    lax.fori_loop(0, CHUNKS // 2, _pair, None)

    plsc.subcore_barrier()
    pltpu.sync_copy(acc_sh.at[pl.ds(s * RPT, RPT)],
                    out_hbm.at[c, pl.ds(s * RPT, RPT)])


def _r_from_deg(degp_ref):
    deg = degp_ref[0, :, 0:1] + degp_ref[1, :, 0:1]
    return lax.rsqrt(jnp.maximum(deg, 1.0))


def _tc_scale_body(degp_ref, x_ref, o_ref):
    o_ref[...] = x_ref[...] * _r_from_deg(degp_ref)


def _tc_layer_body(degp_ref, vp_ref, w_ref, b_ref, o_ref):
    r = _r_from_deg(degp_ref)
    t = (vp_ref[0] + vp_ref[1]) * r
    h = jnp.maximum(jnp.dot(t, w_ref[...],
                            preferred_element_type=jnp.float32) + b_ref[...], 0.0)
    o_ref[...] = h * r


def _tc_head_body(degp_ref, vp_ref, w_ref, b_ref, ws_ref, bs_ref, o_ref):
    r = _r_from_deg(degp_ref)
    t = (vp_ref[0] + vp_ref[1]) * r
    h = jnp.maximum(jnp.dot(t, w_ref[...],
                            preferred_element_type=jnp.float32) + b_ref[...], 0.0)
    logits = jnp.dot(h, ws_ref[...], preferred_element_type=jnp.float32) + bs_ref[...]
    m = jnp.max(logits, axis=-1, keepdims=True)
    e = jnp.exp(logits - m)
    o_ref[...] = e / jnp.sum(e, axis=-1, keepdims=True)


_deg_spec = pl.BlockSpec((NC, RB, D), lambda i: (0, i, 0))
_vp_spec = pl.BlockSpec((NC, RB, D), lambda i: (0, i, 0))
_row_spec = pl.BlockSpec((RB, D), lambda i: (i, 0))
_full = lambda *shape: pl.BlockSpec(shape, lambda i: (0,) * len(shape))


def _tc_scale(degp, xpad):
    return pl.pallas_call(
        _tc_scale_body,
        grid=(NP // RB,),
        in_specs=[_deg_spec, _row_spec],
        out_specs=_row_spec,
        out_shape=jax.ShapeDtypeStruct((NP, D), jnp.float32),
    )(degp, xpad)


def _tc_layer(degp, vp, w, b):
    return pl.pallas_call(
        _tc_layer_body,
        grid=(NP // RB,),
        in_specs=[_deg_spec, _vp_spec, _full(D, D), _full(D)],
        out_specs=_row_spec,
        out_shape=jax.ShapeDtypeStruct((NP, D), jnp.float32),
    )(degp, vp, w, b)


def _tc_head(degp, vp, w, b, ws, bs):
    return pl.pallas_call(
        _tc_head_body,
        grid=(NP // RB,),
        in_specs=[_deg_spec, _vp_spec, _full(D, D), _full(D), _full(D, K), _full(K)],
        out_specs=pl.BlockSpec((RB, K), lambda i: (i, 0)),
        out_shape=jax.ShapeDtypeStruct((NP, K), jnp.float32),
    )(degp, vp, w, b, ws, bs)


def kernel(x, edge_index, W1, b1, W2, b2, Ws, bs):
    src = jnp.concatenate(
        [edge_index[0], jnp.full((EPAD - E,), DUMMY, jnp.int32)])
    dst = jnp.concatenate(
        [edge_index[1], jnp.full((EPAD - E,), DUMMY, jnp.int32)])
    xpad = jnp.pad(x, ((0, NP - N), (0, 0)))

    degp = _sc_hist(dst)
    u1 = _tc_scale(degp, xpad)
    v1 = _sc_spmm(u1, src, dst)
    u2 = _tc_layer(degp, v1, W1, b1)
    v2 = _sc_spmm(u2, src, dst)
    out = _tc_head(degp, v2, W2, b2, Ws, bs)
    return out[:N]


# speedup vs baseline: 1.0673x; 1.0673x over previous
import functools

import jax
import jax.numpy as jnp
from jax import lax
from jax.experimental import pallas as pl
from jax.experimental.pallas import tpu as pltpu
from jax.experimental.pallas import tpu_sc as plsc

N = 10000
E = 320000
D = 128
K = 16

NC = 2
NS = 16
NW = NC * NS
B = 128
NBUF = 2
CHUNKS = 80
EPT = CHUNKS * B
EPAD = EPT * NW
DUMMY = N
NP = 10240
RPT = NP // NS
RB = 1024


_mesh = plsc.VectorSubcoreMesh(core_axis_name="c", subcore_axis_name="s")


@functools.partial(
    pl.kernel,
    mesh=_mesh,
    out_type=jax.ShapeDtypeStruct((NC, NP, D), jnp.float32),
    scratch_types=[
        pltpu.VMEM((B,), jnp.int32),
        pltpu.VMEM((B, D), jnp.float32),
        pltpu.VMEM((B, D), jnp.float32),
        pltpu.VMEM_SHARED((NP, D), jnp.float32),
        pltpu.SemaphoreType.DMA,
    ],
)
def _sc_hist(dst_hbm, out_hbm, idx_v, ones_v, zero_v, deg_sh, sem):
    c = lax.axis_index("c")
    s = lax.axis_index("s")

    def _fill(i, _):
        for j in range(D // 16):
            ones_v[i, pl.ds(j * 16, 16)] = jnp.ones((16,), jnp.float32)
            zero_v[i, pl.ds(j * 16, 16)] = jnp.zeros((16,), jnp.float32)
        return _

    lax.fori_loop(0, B, _fill, None)
    for k in range(RPT // B):
        pltpu.sync_copy(zero_v, deg_sh.at[pl.ds(s * RPT + k * B, B)])
    plsc.subcore_barrier()

    base = (c * NS + s) * EPT

    def _step(ch, _):
        pltpu.sync_copy(dst_hbm.at[pl.ds(base + ch * B, B)], idx_v)
        pltpu.sync_copy(ones_v, deg_sh.at[idx_v], add=True)
        return _

    lax.fori_loop(0, CHUNKS, _step, None)
    plsc.subcore_barrier()
    pltpu.sync_copy(deg_sh.at[pl.ds(s * RPT, RPT)],
                    out_hbm.at[c, pl.ds(s * RPT, RPT)])


@functools.partial(
    pl.kernel,
    mesh=_mesh,
    out_type=jax.ShapeDtypeStruct((NC, NP, D), jnp.float32),
    scratch_types=[
        pltpu.VMEM((2, B), jnp.int32),
        pltpu.VMEM((B, D), jnp.float32),
        pltpu.VMEM_SHARED((NP, D), jnp.float32),
        pltpu.SemaphoreType.DMA,
    ],
)
def _sc_spmm(y_hbm, eidx_hbm, out_hbm, eb_v, rows_v, acc_sh, sem):
    c = lax.axis_index("c")
    s = lax.axis_index("s")
    base = (c * NS + s) * CHUNKS

    def _zero(i, _):
        for j in range(D // 16):
            rows_v[i, pl.ds(j * 16, 16)] = jnp.zeros((16,), jnp.float32)
        return _

    lax.fori_loop(0, B, _zero, None)
    for k in range(RPT // B):
        pltpu.sync_copy(rows_v, acc_sh.at[pl.ds(s * RPT + k * B, B)])
    plsc.subcore_barrier()

    def _step(ch, _):
        pltpu.sync_copy(eidx_hbm.at[base + ch], eb_v)
        pltpu.async_copy(y_hbm.at[eb_v.at[0]], rows_v, sem).wait()
        pltpu.sync_copy(rows_v, acc_sh.at[eb_v.at[1]], add=True)
        return _

    lax.fori_loop(0, CHUNKS, _step, None)

    plsc.subcore_barrier()
    pltpu.sync_copy(acc_sh.at[pl.ds(s * RPT, RPT)],
                    out_hbm.at[c, pl.ds(s * RPT, RPT)])


def _r_from_deg(degp_ref):
    deg = degp_ref[0, :, 0:1] + degp_ref[1, :, 0:1]
    return lax.rsqrt(jnp.maximum(deg, 1.0))


def _tc_scale_body(degp_ref, x_ref, o_ref):
    o_ref[...] = x_ref[...] * _r_from_deg(degp_ref)


def _tc_layer_body(degp_ref, vp_ref, w_ref, b_ref, o_ref):
    r = _r_from_deg(degp_ref)
    t = (vp_ref[0] + vp_ref[1]) * r
    h = jnp.maximum(jnp.dot(t, w_ref[...],
                            preferred_element_type=jnp.float32) + b_ref[...], 0.0)
    o_ref[...] = h * r


def _tc_head_body(degp_ref, vp_ref, w_ref, b_ref, ws_ref, bs_ref, o_ref):
    r = _r_from_deg(degp_ref)
    t = (vp_ref[0] + vp_ref[1]) * r
    h = jnp.maximum(jnp.dot(t, w_ref[...],
                            preferred_element_type=jnp.float32) + b_ref[...], 0.0)
    logits = jnp.dot(h, ws_ref[...], preferred_element_type=jnp.float32) + bs_ref[...]
    m = jnp.max(logits, axis=-1, keepdims=True)
    e = jnp.exp(logits - m)
    o_ref[...] = e / jnp.sum(e, axis=-1, keepdims=True)


_deg_spec = pl.BlockSpec((NC, RB, D), lambda i: (0, i, 0))
_vp_spec = pl.BlockSpec((NC, RB, D), lambda i: (0, i, 0))
_row_spec = pl.BlockSpec((RB, D), lambda i: (i, 0))
_full = lambda *shape: pl.BlockSpec(shape, lambda i: (0,) * len(shape))


def _tc_scale(degp, xpad):
    return pl.pallas_call(
        _tc_scale_body,
        grid=(NP // RB,),
        in_specs=[_deg_spec, _row_spec],
        out_specs=_row_spec,
        out_shape=jax.ShapeDtypeStruct((NP, D), jnp.float32),
    )(degp, xpad)


def _tc_layer(degp, vp, w, b):
    return pl.pallas_call(
        _tc_layer_body,
        grid=(NP // RB,),
        in_specs=[_deg_spec, _vp_spec, _full(D, D), _full(D)],
        out_specs=_row_spec,
        out_shape=jax.ShapeDtypeStruct((NP, D), jnp.float32),
    )(degp, vp, w, b)


def _tc_head(degp, vp, w, b, ws, bs):
    return pl.pallas_call(
        _tc_head_body,
        grid=(NP // RB,),
        in_specs=[_deg_spec, _vp_spec, _full(D, D), _full(D), _full(D, K), _full(K)],
        out_specs=pl.BlockSpec((RB, K), lambda i: (i, 0)),
        out_shape=jax.ShapeDtypeStruct((NP, K), jnp.float32),
    )(degp, vp, w, b, ws, bs)


def kernel(x, edge_index, W1, b1, W2, b2, Ws, bs):
    src = jnp.concatenate(
        [edge_index[0], jnp.full((EPAD - E,), DUMMY, jnp.int32)])
    dst = jnp.concatenate(
        [edge_index[1], jnp.full((EPAD - E,), DUMMY, jnp.int32)])
    eidx = jnp.stack(
        [src.reshape(NW * CHUNKS, B), dst.reshape(NW * CHUNKS, B)], axis=1)
    xpad = jnp.pad(x, ((0, NP - N), (0, 0)))

    degp = _sc_hist(dst)
    u1 = _tc_scale(degp, xpad)
    v1 = _sc_spmm(u1, eidx)
    u2 = _tc_layer(degp, v1, W1, b1)
    v2 = _sc_spmm(u2, eidx)
    out = _tc_head(degp, v2, W2, b2, Ws, bs)
    return out[:N]


# speedup vs baseline: 2.1584x; 2.0223x over previous
import functools

import jax
import jax.numpy as jnp
from jax import lax
from jax.experimental import pallas as pl
from jax.experimental.pallas import tpu as pltpu
from jax.experimental.pallas import tpu_sc as plsc

N = 10000
E = 320000
D = 128
K = 16

NC = 2
NS = 16
NW = NC * NS
B = 128
NBUF = 2
CHUNKS = 80
EPT = CHUNKS * B
EPAD = EPT * NW
DUMMY = N
NP = 10240
RPT = NP // NS
RB = 1024


_mesh = plsc.VectorSubcoreMesh(core_axis_name="c", subcore_axis_name="s")


@functools.partial(
    pl.kernel,
    mesh=_mesh,
    out_type=jax.ShapeDtypeStruct((NC, NP, D), jnp.float32),
    scratch_types=[
        pltpu.VMEM((B,), jnp.int32),
        pltpu.VMEM((B, D), jnp.float32),
        pltpu.VMEM((B, D), jnp.float32),
        pltpu.VMEM_SHARED((NP, D), jnp.float32),
        pltpu.SemaphoreType.DMA,
    ],
)
def _sc_hist(dst_hbm, out_hbm, idx_v, ones_v, zero_v, deg_sh, sem):
    c = lax.axis_index("c")
    s = lax.axis_index("s")

    def _fill(i, _):
        for j in range(D // 16):
            ones_v[i, pl.ds(j * 16, 16)] = jnp.ones((16,), jnp.float32)
            zero_v[i, pl.ds(j * 16, 16)] = jnp.zeros((16,), jnp.float32)
        return _

    lax.fori_loop(0, B, _fill, None)
    for k in range(RPT // B):
        pltpu.sync_copy(zero_v, deg_sh.at[pl.ds(s * RPT + k * B, B)])
    plsc.subcore_barrier()

    base = (c * NS + s) * EPT

    def _step(ch, _):
        pltpu.sync_copy(dst_hbm.at[pl.ds(base + ch * B, B)], idx_v)
        pltpu.sync_copy(ones_v, deg_sh.at[idx_v], add=True)
        return _

    lax.fori_loop(0, CHUNKS, _step, None)
    plsc.subcore_barrier()
    pltpu.sync_copy(deg_sh.at[pl.ds(s * RPT, RPT)],
                    out_hbm.at[c, pl.ds(s * RPT, RPT)])


@functools.partial(
    pl.kernel,
    mesh=_mesh,
    out_type=jax.ShapeDtypeStruct((NC, NP, D), jnp.float32),
    scratch_types=[
        pltpu.VMEM((2, B), jnp.int32),
        pltpu.VMEM((B, D), jnp.float32),
        pltpu.VMEM_SHARED((NP, D), jnp.float32),
        pltpu.SemaphoreType.DMA,
    ],
)
def _sc_spmm(y_hbm, eidx_hbm, out_hbm, eb_v, rows_v, acc_sh, sem):
    c = lax.axis_index("c")
    s = lax.axis_index("s")
    base = (c * NS + s) * CHUNKS

    def _zero(i, _):
        for j in range(D // 16):
            rows_v[i, pl.ds(j * 16, 16)] = jnp.zeros((16,), jnp.float32)
        return _

    lax.fori_loop(0, B, _zero, None)
    for k in range(RPT // B):
        pltpu.sync_copy(rows_v, acc_sh.at[pl.ds(s * RPT + k * B, B)])
    plsc.subcore_barrier()

    def _step(ch, _):
        pltpu.sync_copy(eidx_hbm.at[base + ch], eb_v)
        pltpu.async_copy(y_hbm.at[eb_v.at[0]], rows_v, sem).wait()
        pltpu.sync_copy(rows_v, acc_sh.at[eb_v.at[1]], add=True)
        return _

    lax.fori_loop(0, CHUNKS, _step, None)

    plsc.subcore_barrier()
    pltpu.sync_copy(acc_sh.at[pl.ds(s * RPT, RPT)],
                    out_hbm.at[c, pl.ds(s * RPT, RPT)])


def _r_from_deg(degp_ref):
    deg = degp_ref[0, :, 0:1] + degp_ref[1, :, 0:1]
    return lax.rsqrt(jnp.maximum(deg, 1.0))


def _tc_scale_body(degp_ref, x_ref, o_ref):
    o_ref[...] = x_ref[...] * _r_from_deg(degp_ref)


def _tc_layer_body(degp_ref, vp_ref, w_ref, b_ref, o_ref):
    r = _r_from_deg(degp_ref)
    t = (vp_ref[0] + vp_ref[1]) * r
    h = jnp.maximum(jnp.dot(t, w_ref[...],
                            preferred_element_type=jnp.float32) + b_ref[...], 0.0)
    o_ref[...] = h * r


def _tc_head_body(degp_ref, vp_ref, w_ref, b_ref, ws_ref, bs_ref, o_ref):
    r = _r_from_deg(degp_ref)
    t = (vp_ref[0] + vp_ref[1]) * r
    h = jnp.maximum(jnp.dot(t, w_ref[...],
                            preferred_element_type=jnp.float32) + b_ref[...], 0.0)
    logits = jnp.dot(h, ws_ref[...], preferred_element_type=jnp.float32) + bs_ref[...]
    m = jnp.max(logits, axis=-1, keepdims=True)
    e = jnp.exp(logits - m)
    o_ref[...] = e / jnp.sum(e, axis=-1, keepdims=True)


_deg_spec = pl.BlockSpec((NC, RB, D), lambda i: (0, i, 0))
_vp_spec = pl.BlockSpec((NC, RB, D), lambda i: (0, i, 0))
_row_spec = pl.BlockSpec((RB, D), lambda i: (i, 0))
_full = lambda *shape: pl.BlockSpec(shape, lambda i: (0,) * len(shape))


def _tc_scale(degp, xpad):
    return pl.pallas_call(
        _tc_scale_body,
        grid=(NP // RB,),
        in_specs=[_deg_spec, _row_spec],
        out_specs=_row_spec,
        out_shape=jax.ShapeDtypeStruct((NP, D), jnp.float32),
    )(degp, xpad)


def _tc_layer(degp, vp, w, b):
    return pl.pallas_call(
        _tc_layer_body,
        grid=(NP // RB,),
        in_specs=[_deg_spec, _vp_spec, _full(D, D), _full(D)],
        out_specs=_row_spec,
        out_shape=jax.ShapeDtypeStruct((NP, D), jnp.float32),
    )(degp, vp, w, b)


def _tc_head(degp, vp, w, b, ws, bs):
    return pl.pallas_call(
        _tc_head_body,
        grid=(NP // RB,),
        in_specs=[_deg_spec, _vp_spec, _full(D, D), _full(D), _full(D, K), _full(K)],
        out_specs=pl.BlockSpec((RB, K), lambda i: (i, 0)),
        out_shape=jax.ShapeDtypeStruct((NP, K), jnp.float32),
    )(degp, vp, w, b, ws, bs)


def kernel(x, edge_index, W1, b1, W2, b2, Ws, bs):
    pad_idx = DUMMY + (jnp.arange(EPAD - E, dtype=jnp.int32) % (NP - N))
    src = jnp.concatenate([edge_index[0], pad_idx])
    dst = jnp.concatenate([edge_index[1], pad_idx])
    eidx = jnp.stack(
        [src.reshape(NW * CHUNKS, B), dst.reshape(NW * CHUNKS, B)], axis=1)
    xpad = jnp.pad(x, ((0, NP - N), (0, 0)))

    degp = _sc_hist(dst)
    u1 = _tc_scale(degp, xpad)
    v1 = _sc_spmm(u1, eidx)
    u2 = _tc_layer(degp, v1, W1, b1)
    v2 = _sc_spmm(u2, eidx)
    out = _tc_head(degp, v2, W2, b2, Ws, bs)
    return out[:N]


# speedup vs baseline: 2.5717x; 1.1915x over previous
import functools

import jax
import jax.numpy as jnp
from jax import lax
from jax.experimental import pallas as pl
from jax.experimental.pallas import tpu as pltpu
from jax.experimental.pallas import tpu_sc as plsc

N = 10000
E = 320000
D = 128
K = 16

NC = 2
NS = 16
NW = NC * NS
B = 128
NBUF = 2
CHUNKS = 80
EPT = CHUNKS * B
EPAD = EPT * NW
DUMMY = N
NP = 10240
RPT = NP // NS
RB = 1024


_mesh = plsc.VectorSubcoreMesh(core_axis_name="c", subcore_axis_name="s")


@functools.partial(
    pl.kernel,
    mesh=_mesh,
    out_type=jax.ShapeDtypeStruct((NW, NP), jnp.float32),
    scratch_types=[
        pltpu.VMEM((EPT,), jnp.int32),
        pltpu.VMEM((NP,), jnp.float32),
    ],
    compiler_params=pltpu.CompilerParams(needs_layout_passes=False),
)
def _sc_hist(dst_hbm, out_hbm, dstw_v, hist_v):
    c = lax.axis_index("c")
    s = lax.axis_index("s")
    wid = c * NS + s

    def _zh(i, _):
        hist_v[pl.ds(i * 16, 16)] = jnp.zeros((16,), jnp.float32)
        return _

    lax.fori_loop(0, NP // 16, _zh, None)
    pltpu.sync_copy(dst_hbm.at[wid], dstw_v)
    ones = jnp.ones((16,), jnp.float32)

    def _e(k, _):
        idx = dstw_v[pl.ds(k * 16, 16)]
        plsc.addupdate_scatter(hist_v, [idx], ones)
        return _

    lax.fori_loop(0, EPT // 16, _e, None)
    pltpu.sync_copy(hist_v, out_hbm.at[wid])


@functools.partial(
    pl.kernel,
    mesh=_mesh,
    out_type=jax.ShapeDtypeStruct((NC, NP, D), jnp.float32),
    scratch_types=[
        pltpu.VMEM((2, B), jnp.int32),
        pltpu.VMEM((B, D), jnp.float32),
        pltpu.VMEM_SHARED((NP, D), jnp.float32),
        pltpu.SemaphoreType.DMA,
    ],
)
def _sc_spmm(y_hbm, eidx_hbm, out_hbm, eb_v, rows_v, acc_sh, sem):
    c = lax.axis_index("c")
    s = lax.axis_index("s")
    base = (c * NS + s) * CHUNKS

    def _zero(i, _):
        for j in range(D // 16):
            rows_v[i, pl.ds(j * 16, 16)] = jnp.zeros((16,), jnp.float32)
        return _

    lax.fori_loop(0, B, _zero, None)
    for k in range(RPT // B):
        pltpu.sync_copy(rows_v, acc_sh.at[pl.ds(s * RPT + k * B, B)])
    plsc.subcore_barrier()

    def _step(ch, _):
        pltpu.sync_copy(eidx_hbm.at[base + ch], eb_v)
        pltpu.async_copy(y_hbm.at[eb_v.at[0]], rows_v, sem).wait()
        pltpu.sync_copy(rows_v, acc_sh.at[eb_v.at[1]], add=True)
        return _

    lax.fori_loop(0, CHUNKS, _step, None)

    plsc.subcore_barrier()
    pltpu.sync_copy(acc_sh.at[pl.ds(s * RPT, RPT)],
                    out_hbm.at[c, pl.ds(s * RPT, RPT)])


def _r_from_deg(degp_ref):
    deg = jnp.sum(degp_ref[...], axis=0)[:, None]
    return lax.rsqrt(jnp.maximum(deg, 1.0))


def _tc_scale_body(degp_ref, x_ref, o_ref):
    o_ref[...] = x_ref[...] * _r_from_deg(degp_ref)


def _tc_layer_body(degp_ref, vp_ref, w_ref, b_ref, o_ref):
    r = _r_from_deg(degp_ref)
    t = (vp_ref[0] + vp_ref[1]) * r
    h = jnp.maximum(jnp.dot(t, w_ref[...],
                            preferred_element_type=jnp.float32) + b_ref[...], 0.0)
    o_ref[...] = h * r


def _tc_head_body(degp_ref, vp_ref, w_ref, b_ref, ws_ref, bs_ref, o_ref):
    r = _r_from_deg(degp_ref)
    t = (vp_ref[0] + vp_ref[1]) * r
    h = jnp.maximum(jnp.dot(t, w_ref[...],
                            preferred_element_type=jnp.float32) + b_ref[...], 0.0)
    logits = jnp.dot(h, ws_ref[...], preferred_element_type=jnp.float32) + bs_ref[...]
    m = jnp.max(logits, axis=-1, keepdims=True)
    e = jnp.exp(logits - m)
    o_ref[...] = e / jnp.sum(e, axis=-1, keepdims=True)


_deg_spec = pl.BlockSpec((NW, RB), lambda i: (0, i))
_vp_spec = pl.BlockSpec((NC, RB, D), lambda i: (0, i, 0))
_row_spec = pl.BlockSpec((RB, D), lambda i: (i, 0))
_full = lambda *shape: pl.BlockSpec(shape, lambda i: (0,) * len(shape))


def _tc_scale(degp, xpad):
    return pl.pallas_call(
        _tc_scale_body,
        grid=(NP // RB,),
        in_specs=[_deg_spec, _row_spec],
        out_specs=_row_spec,
        out_shape=jax.ShapeDtypeStruct((NP, D), jnp.float32),
    )(degp, xpad)


def _tc_layer(degp, vp, w, b):
    return pl.pallas_call(
        _tc_layer_body,
        grid=(NP // RB,),
        in_specs=[_deg_spec, _vp_spec, _full(D, D), _full(D)],
        out_specs=_row_spec,
        out_shape=jax.ShapeDtypeStruct((NP, D), jnp.float32),
    )(degp, vp, w, b)


def _tc_head(degp, vp, w, b, ws, bs):
    return pl.pallas_call(
        _tc_head_body,
        grid=(NP // RB,),
        in_specs=[_deg_spec, _vp_spec, _full(D, D), _full(D), _full(D, K), _full(K)],
        out_specs=pl.BlockSpec((RB, K), lambda i: (i, 0)),
        out_shape=jax.ShapeDtypeStruct((NP, K), jnp.float32),
    )(degp, vp, w, b, ws, bs)


def kernel(x, edge_index, W1, b1, W2, b2, Ws, bs):
    pad_idx = DUMMY + (jnp.arange(EPAD - E, dtype=jnp.int32) % (NP - N))
    src = jnp.concatenate([edge_index[0], pad_idx])
    dst = jnp.concatenate([edge_index[1], pad_idx])
    eidx = jnp.stack(
        [src.reshape(NW * CHUNKS, B), dst.reshape(NW * CHUNKS, B)], axis=1)
    xpad = jnp.pad(x, ((0, NP - N), (0, 0)))

    degp = _sc_hist(dst.reshape(NW, EPT))
    u1 = _tc_scale(degp, xpad)
    v1 = _sc_spmm(u1, eidx)
    u2 = _tc_layer(degp, v1, W1, b1)
    v2 = _sc_spmm(u2, eidx)
    out = _tc_head(degp, v2, W2, b2, Ws, bs)
    return out[:N]
